# Initial kernel scaffold; baseline (speedup 1.0000x reference)
import jax
import jax.numpy as jnp
from jax.experimental import pallas as pl


def kernel(x, edge_index, idx, edge_type, params):
    raise NotImplementedError("write your pallas kernel here")


# speedup vs baseline: 4.2501x; 4.2501x over previous
import functools

import jax
import jax.numpy as jnp
from jax import lax
from jax.experimental import pallas as pl
from jax.experimental.pallas import tpu as pltpu
from jax.experimental.pallas import tpu_sc as plsc

N = 3976
E = 254464
D = 128
REL = 4
IDXN = 1024

NW = 32
FPW = D // NW
NP = 3984
CHUNK = 512
VPC = CHUNK // 16
NCHUNK = E // CHUNK
EPW = E // NW

_MESH = plsc.VectorSubcoreMesh(core_axis_name="c", subcore_axis_name="s")
_SC_PARAMS = pltpu.CompilerParams(needs_layout_passes=False)


def _wid():
    return lax.axis_index("s") * 2 + lax.axis_index("c")


@functools.partial(
    pl.kernel,
    mesh=_MESH,
    compiler_params=_SC_PARAMS,
    out_type=jax.ShapeDtypeStruct((NW, NP), jnp.float32),
    scratch_types=[
        pltpu.VMEM((NP,), jnp.float32),
        pltpu.VMEM((EPW,), jnp.int32),
    ],
)
def _deg_kernel(dst_hbm, out, deg_v, dstb):
    w = _wid()
    zero = jnp.zeros((16,), jnp.float32)

    def zbody(i, c):
        deg_v[pl.ds(i * 16, 16)] = zero
        return c

    lax.fori_loop(0, NP // 16, zbody, 0)
    pltpu.sync_copy(dst_hbm.at[pl.ds(w * EPW, EPW)], dstb)
    ones = jnp.full((16,), 1.0, jnp.float32)

    def body(v, c):
        dv = dstb[pl.ds(v * 16, 16)]
        plsc.addupdate_scatter(deg_v, [dv], ones)
        return c

    lax.fori_loop(0, EPW // 16, body, 0)
    pltpu.sync_copy(deg_v, out.at[w])


@functools.partial(
    pl.kernel,
    mesh=_MESH,
    compiler_params=_SC_PARAMS,
    out_type=[
        jax.ShapeDtypeStruct((D, NP), jnp.float32),
        jax.ShapeDtypeStruct((D, NP), jnp.float32),
        jax.ShapeDtypeStruct((D, NP), jnp.float32),
    ],
    scratch_types=[
        pltpu.VMEM((FPW, REL * N), jnp.float32),
        pltpu.VMEM((FPW, N), jnp.float32),
        pltpu.VMEM((FPW, NP), jnp.float32),
        pltpu.VMEM((FPW, NP), jnp.float32),
        pltpu.VMEM((FPW, NP), jnp.float32),
        pltpu.VMEM((CHUNK,), jnp.int32),
        pltpu.VMEM((CHUNK,), jnp.int32),
    ],
)
def _edge_kernel(t_hbm, y_hbm, src_hbm, key_hbm, s_out, num_out, g_out,
                 t_v, y_v, s_v, num_v, g_v, srcb, keyb):
    w = _wid()
    fbase = w * FPW
    pltpu.sync_copy(t_hbm.at[pl.ds(fbase, FPW)], t_v)
    pltpu.sync_copy(y_hbm.at[pl.ds(fbase, FPW)], y_v)

    zero = jnp.zeros((16,), jnp.float32)

    def zbody(i, c):
        for f in range(FPW):
            s_v[f, pl.ds(i * 16, 16)] = zero
            num_v[f, pl.ds(i * 16, 16)] = zero
            g_v[f, pl.ds(i * 16, 16)] = zero
        return c

    lax.fori_loop(0, NP // 16, zbody, 0)

    fvecs = [jnp.full((16,), f, jnp.int32) for f in range(FPW)]

    def body(ci, c):
        base = ci * CHUNK
        pltpu.sync_copy(src_hbm.at[pl.ds(base, CHUNK)], srcb)
        pltpu.sync_copy(key_hbm.at[pl.ds(base, CHUNK)], keyb)
        for v in range(VPC):
            sv = srcb[pl.ds(v * 16, 16)]
            kv = keyb[pl.ds(v * 16, 16)]
            dv = lax.shift_right_logical(kv, 2)
            for f in range(FPW):
                t = plsc.load_gather(t_v, [fvecs[f], kv])
                e = jnp.exp(t)
                plsc.addupdate_scatter(s_v, [fvecs[f], sv], e)
                plsc.addupdate_scatter(num_v, [fvecs[f], sv], t * e)
                yv = plsc.load_gather(y_v, [fvecs[f], dv])
                plsc.addupdate_scatter(g_v, [fvecs[f], sv], yv)
        return c

    lax.fori_loop(0, NCHUNK, body, 0)

    pltpu.sync_copy(s_v, s_out.at[pl.ds(fbase, FPW)])
    pltpu.sync_copy(num_v, num_out.at[pl.ds(fbase, FPW)])
    pltpu.sync_copy(g_v, g_out.at[pl.ds(fbase, FPW)])


_ROWS = IDXN // NW


@functools.partial(
    pl.kernel,
    mesh=_MESH,
    compiler_params=_SC_PARAMS,
    out_type=jax.ShapeDtypeStruct((IDXN, D), jnp.float32),
    scratch_types=[
        pltpu.VMEM((_ROWS,), jnp.int32),
        pltpu.VMEM((_ROWS, D), jnp.float32),
        pltpu.SemaphoreType.DMA,
    ],
)
def _gather_kernel(h_hbm, idx_hbm, out, idx_v, rows_v, sem):
    w = _wid()
    base = w * _ROWS
    pltpu.sync_copy(idx_hbm.at[pl.ds(base, _ROWS)], idx_v)
    pltpu.async_copy(h_hbm.at[idx_v], rows_v, sem).wait()
    pltpu.sync_copy(rows_v, out.at[pl.ds(base, _ROWS)])


def _mm(a, b):
    return jax.lax.dot_general(a, b, (((1,), (0,)), ((), ())),
                               preferred_element_type=jnp.float32)


def _ln_T(tT, g_col, b_col):
    mu = tT.mean(0, keepdims=True)
    var = ((tT - mu) ** 2).mean(0, keepdims=True)
    return (tT - mu) * jax.lax.rsqrt(var + 1e-5) * g_col + b_col


def _dense_stage(xT, dinv_row, wrel_t, pg_wt, pg_b, ng_g, ng_b):
    gT = _ln_T(jax.nn.relu(_mm(pg_wt, xT) + pg_b), ng_g, ng_b)
    yT = xT * dinv_row
    ps = [_mm(wrel_t[r], xT) for r in range(REL)]
    m = ps[0].max(1, keepdims=True)
    for r in range(1, REL):
        m = jnp.maximum(m, ps[r].max(1, keepdims=True))
    ps = [p - m for p in ps]
    return gT, yT, ps, m


def _pre_body(x0t_ref, deg_ref, proj_wt, proj_b, bn_g, bn_b,
              wi_wt, wi_b, pg_wt, pg_b, ng_g, ng_b, wrel_t,
              gT_o, yT_o, p0_o, p1_o, p2_o, p3_o, m_o):
    x0t = x0t_ref[...]
    deg = jnp.sum(deg_ref[...], axis=0)[:N]
    dinv_row = jnp.where(deg > 0, jax.lax.rsqrt(jnp.maximum(deg, 1.0)), 0.0)[None, :]
    hp = _mm(proj_wt[...], x0t) + proj_b[...]
    mu = hp.mean(1, keepdims=True)
    var = ((hp - mu) ** 2).mean(1, keepdims=True)
    hT = jax.nn.relu((hp - mu) * jax.lax.rsqrt(var + 1e-5) * bn_g[...] + bn_b[...])
    xT = _mm(wi_wt[...], hT) + wi_b[...]
    gT, yT, ps, m = _dense_stage(xT, dinv_row, wrel_t[...], pg_wt[...],
                                 pg_b[...], ng_g[...], ng_b[...])
    gT_o[...] = gT
    yT_o[...] = yT
    p0_o[...], p1_o[...], p2_o[...], p3_o[...] = ps
    m_o[...] = m


def _combine(gT, s_p, num_p, g2_p, m, dinv_row, co_wt, co_b, n_g, n_b):
    s = s_p[:, :N]
    num = num_p[:, :N]
    g2 = g2_p[:, :N]
    msg = jnp.where(s > 0, num / jnp.maximum(s, 1e-37) + m, 0.0)
    totT = gT + g2 * dinv_row + 0.1 * jax.nn.relu(msg)
    return _ln_T(_mm(co_wt, totT) + co_b, n_g, n_b)


def _mid_body(gT_ref, s_ref, num_ref, g2_ref, m_ref, deg_ref,
              co_wt, co_b, n_g, n_b,
              wi_wt, wi_b, pg_wt, pg_b, ng_g, ng_b, wrel_t,
              gT_o, yT_o, p0_o, p1_o, p2_o, p3_o, m_o):
    deg = jnp.sum(deg_ref[...], axis=0)[:N]
    dinv_row = jnp.where(deg > 0, jax.lax.rsqrt(jnp.maximum(deg, 1.0)), 0.0)[None, :]
    hT = _combine(gT_ref[...], s_ref[...], num_ref[...], g2_ref[...],
                  m_ref[...], dinv_row, co_wt[...], co_b[...], n_g[...], n_b[...])
    xT = _mm(wi_wt[...], hT) + wi_b[...]
    gT, yT, ps, m = _dense_stage(xT, dinv_row, wrel_t[...], pg_wt[...],
                                 pg_b[...], ng_g[...], ng_b[...])
    gT_o[...] = gT
    yT_o[...] = yT
    p0_o[...], p1_o[...], p2_o[...], p3_o[...] = ps
    m_o[...] = m


def _post_body(gT_ref, s_ref, num_ref, g2_ref, m_ref, deg_ref,
               co_wt, co_b, n_g, n_b, h_o):
    deg = jnp.sum(deg_ref[...], axis=0)[:N]
    dinv_row = jnp.where(deg > 0, jax.lax.rsqrt(jnp.maximum(deg, 1.0)), 0.0)[None, :]
    hT = _combine(gT_ref[...], s_ref[...], num_ref[...], g2_ref[...],
                  m_ref[...], dinv_row, co_wt[...], co_b[...], n_g[...], n_b[...])
    h_o[...] = hT * 0.5 * (1.0 + jax.lax.erf(hT * (2.0 ** -0.5)))


_TN = jax.ShapeDtypeStruct((D, N), jnp.float32)
_DENSE_OUT = [_TN, _TN, _TN, _TN, _TN, _TN,
              jax.ShapeDtypeStruct((D, 1), jnp.float32)]

_pre_call = pl.pallas_call(_pre_body, out_shape=_DENSE_OUT)
_mid_call = pl.pallas_call(_mid_body, out_shape=_DENSE_OUT)
_post_call = pl.pallas_call(_post_body, out_shape=_TN)


def _interleave(ps):
    return jnp.stack(ps, axis=2).reshape(D, REL * N)


def kernel(x, edge_index, idx, edge_type, params):
    src = edge_index[0]
    dst = edge_index[1]
    key2 = dst * 4 + edge_type

    deg_part = _deg_kernel(dst)

    l1, l2 = params["layers"]

    def col(v):
        return v.reshape(D, 1)

    gT, yT, p0, p1, p2, p3, m1 = _pre_call(
        x.T, deg_part, params["proj_w"].T, col(params["proj_b"]),
        col(params["bn_g"]), col(params["bn_b"]),
        l1["wi_w"].T, col(l1["wi_b"]), l1["pg_w"].T, col(l1["pg_b"]),
        col(l1["ng_g"]), col(l1["ng_b"]),
        jnp.swapaxes(l1["w_rel"], 1, 2))

    s1, num1, g21 = _edge_kernel(_interleave([p0, p1, p2, p3]), yT, src, key2)

    gT2, yT2, q0, q1, q2, q3, m2 = _mid_call(
        gT, s1, num1, g21, m1, deg_part,
        l1["co_w"].T, col(l1["co_b"]), col(l1["n_g"]), col(l1["n_b"]),
        l2["wi_w"].T, col(l2["wi_b"]), l2["pg_w"].T, col(l2["pg_b"]),
        col(l2["ng_g"]), col(l2["ng_b"]),
        jnp.swapaxes(l2["w_rel"], 1, 2))

    s2, num2, g22 = _edge_kernel(_interleave([q0, q1, q2, q3]), yT2, src, key2)

    hT = _post_call(gT2, s2, num2, g22, m2, deg_part,
                    l2["co_w"].T, col(l2["co_b"]), col(l2["n_g"]), col(l2["n_b"]))

    return _gather_kernel(hT.T, idx)


# speedup vs baseline: 15.4511x; 3.6355x over previous
import functools

import jax
import jax.numpy as jnp
from jax import lax
from jax.experimental import pallas as pl
from jax.experimental.pallas import tpu as pltpu
from jax.experimental.pallas import tpu_sc as plsc

N = 3976
E = 254464
D = 128
REL = 4
IDXN = 1024

NW = 32
NP = 4096
RPS = NP // 16
ZR = RPS // 8
C = 56
EPW = E // NW
NCH = EPW // C

_MESH = plsc.VectorSubcoreMesh(core_axis_name="c", subcore_axis_name="s")
_SC_PARAMS = pltpu.CompilerParams(needs_layout_passes=False)


@functools.partial(
    pl.kernel,
    mesh=_MESH,
    compiler_params=_SC_PARAMS,
    out_type=jax.ShapeDtypeStruct((NW, NP), jnp.float32),
    scratch_types=[
        pltpu.VMEM((NP,), jnp.float32),
        pltpu.VMEM((EPW,), jnp.int32),
    ],
)
def _deg_kernel(dst_hbm, out, deg_v, dstb):
    w = lax.axis_index("s") * 2 + lax.axis_index("c")
    zero = jnp.zeros((16,), jnp.float32)

    def zbody(i, c):
        deg_v[pl.ds(i * 16, 16)] = zero
        return c

    lax.fori_loop(0, NP // 16, zbody, 0)
    pltpu.sync_copy(dst_hbm.at[pl.ds(w * EPW, EPW)], dstb)
    ones = jnp.full((16,), 1.0, jnp.float32)

    def body(v, c):
        dv = dstb[pl.ds(v * 16, 16)]
        plsc.addupdate_scatter(deg_v, [dv], ones)
        return c

    lax.fori_loop(0, EPW // 16, body, 0)
    pltpu.sync_copy(deg_v, out.at[w])


@functools.partial(
    pl.kernel,
    mesh=_MESH,
    compiler_params=_SC_PARAMS,
    out_type=[
        jax.ShapeDtypeStruct((2, NP, D), jnp.float32),
        jax.ShapeDtypeStruct((2, NP, D), jnp.float32),
        jax.ShapeDtypeStruct((2, NP, D), jnp.float32),
    ],
    scratch_types=[
        pltpu.VMEM_SHARED((NP, D), jnp.float32),
        pltpu.VMEM_SHARED((NP, D), jnp.float32),
        pltpu.VMEM_SHARED((NP, D), jnp.float32),
        pltpu.VMEM((C,), jnp.int32),
        pltpu.VMEM((C,), jnp.int32),
        pltpu.VMEM((C,), jnp.int32),
        pltpu.VMEM((C, D), jnp.float32),
        pltpu.VMEM((C, D), jnp.float32),
        pltpu.VMEM((C, D), jnp.float32),
        pltpu.VMEM((ZR, D), jnp.float32),
        pltpu.SemaphoreType.DMA,
        pltpu.SemaphoreType.DMA,
    ],
)
def _edge_kernel(t_hbm, y_hbm, src_hbm, key_hbm, dst_hbm, s_out, num_out, g_out,
                 s_sh, num_sh, g_sh, srcb, keyb, dstb, rowsT, exb, rowsY,
                 zbuf, semT, semY):
    cid = lax.axis_index("c")
    sid = lax.axis_index("s")
    w = sid * 2 + cid
    zero = jnp.zeros((16,), jnp.float32)

    def zb(i, c):
        for f in range(D // 16):
            zbuf[i, pl.ds(f * 16, 16)] = zero
        return c

    lax.fori_loop(0, ZR, zb, 0)
    rbase = sid * RPS
    for j in range(RPS // ZR):
        pltpu.sync_copy(zbuf, s_sh.at[pl.ds(rbase + j * ZR, ZR)])
        pltpu.sync_copy(zbuf, num_sh.at[pl.ds(rbase + j * ZR, ZR)])
        pltpu.sync_copy(zbuf, g_sh.at[pl.ds(rbase + j * ZR, ZR)])
    plsc.subcore_barrier()

    wbase = w * EPW

    def body(ci, c):
        base = wbase + ci * C
        pltpu.sync_copy(src_hbm.at[pl.ds(base, C)], srcb)
        pltpu.sync_copy(key_hbm.at[pl.ds(base, C)], keyb)
        pltpu.sync_copy(dst_hbm.at[pl.ds(base, C)], dstb)
        cpT = pltpu.async_copy(t_hbm.at[keyb], rowsT, semT)
        cpY = pltpu.async_copy(y_hbm.at[dstb], rowsY, semY)
        cpT.wait()

        def crow(r, c2):
            for f in range(D // 16):
                t = rowsT[r, pl.ds(f * 16, 16)]
                e = jnp.exp(t)
                exb[r, pl.ds(f * 16, 16)] = e
                rowsT[r, pl.ds(f * 16, 16)] = t * e
            return c2

        lax.fori_loop(0, C, crow, 0)
        cpY.wait()
        pltpu.sync_copy(exb, s_sh.at[srcb], add=True)
        pltpu.sync_copy(rowsT, num_sh.at[srcb], add=True)
        pltpu.sync_copy(rowsY, g_sh.at[srcb], add=True)
        return c

    lax.fori_loop(0, NCH, body, 0)
    plsc.subcore_barrier()

    pltpu.sync_copy(s_sh.at[pl.ds(rbase, RPS)], s_out.at[cid, pl.ds(rbase, RPS)])
    pltpu.sync_copy(num_sh.at[pl.ds(rbase, RPS)], num_out.at[cid, pl.ds(rbase, RPS)])
    pltpu.sync_copy(g_sh.at[pl.ds(rbase, RPS)], g_out.at[cid, pl.ds(rbase, RPS)])


_ROWS = IDXN // NW


@functools.partial(
    pl.kernel,
    mesh=_MESH,
    compiler_params=_SC_PARAMS,
    out_type=jax.ShapeDtypeStruct((IDXN, D), jnp.float32),
    scratch_types=[
        pltpu.VMEM((_ROWS,), jnp.int32),
        pltpu.VMEM((_ROWS, D), jnp.float32),
        pltpu.SemaphoreType.DMA,
    ],
)
def _gather_kernel(h_hbm, idx_hbm, out, idx_v, rows_v, sem):
    w = lax.axis_index("s") * 2 + lax.axis_index("c")
    base = w * _ROWS
    pltpu.sync_copy(idx_hbm.at[pl.ds(base, _ROWS)], idx_v)
    pltpu.async_copy(h_hbm.at[idx_v], rows_v, sem).wait()
    pltpu.sync_copy(rows_v, out.at[pl.ds(base, _ROWS)])


def _ln(t, g_row, b_row):
    mu = t.mean(1, keepdims=True)
    var = ((t - mu) ** 2).mean(1, keepdims=True)
    return (t - mu) * jax.lax.rsqrt(var + 1e-5) * g_row + b_row


def _dinv_col(deg_ref):
    deg = jnp.sum(deg_ref[...], axis=0)[:N]
    return jnp.where(deg > 0, jax.lax.rsqrt(jnp.maximum(deg, 1.0)), 0.0)[:, None]


def _dense_stage(x, dinv_col, w_rel, pg_w, pg_b, ng_g, ng_b, tab_o):
    g = _ln(jax.nn.relu(x @ pg_w + pg_b), ng_g, ng_b)
    y = x * dinv_col
    ps = [jnp.dot(x, w_rel[r], preferred_element_type=jnp.float32)
          for r in range(REL)]
    m = ps[0].max(0, keepdims=True)
    for r in range(1, REL):
        m = jnp.maximum(m, ps[r].max(0, keepdims=True))
    for r in range(REL):
        tab_o[:, r, :] = ps[r] - m
    return g, y, m


def _pre_body(x0_ref, deg_ref, proj_w, proj_b, bn_g, bn_b,
              wi_w, wi_b, pg_w, pg_b, ng_g, ng_b, w_rel,
              g_o, y_o, tab_o, m_o):
    dinv_col = _dinv_col(deg_ref)
    hp = x0_ref[...] @ proj_w[...] + proj_b[...]
    mu = hp.mean(0, keepdims=True)
    var = ((hp - mu) ** 2).mean(0, keepdims=True)
    h = jax.nn.relu((hp - mu) * jax.lax.rsqrt(var + 1e-5) * bn_g[...] + bn_b[...])
    x = h @ wi_w[...] + wi_b[...]
    g, y, m = _dense_stage(x, dinv_col, w_rel[...], pg_w[...], pg_b[...],
                           ng_g[...], ng_b[...], tab_o)
    g_o[...] = g
    y_o[...] = y
    m_o[...] = m


def _combine(g, s_p, num_p, g2_p, m, dinv_col, co_w, co_b, n_g, n_b):
    s = s_p[0, :N, :] + s_p[1, :N, :]
    num = num_p[0, :N, :] + num_p[1, :N, :]
    g2 = g2_p[0, :N, :] + g2_p[1, :N, :]
    msg = jnp.where(s > 0, num / jnp.maximum(s, 1e-37) + m, 0.0)
    tot = g + g2 * dinv_col + 0.1 * jax.nn.relu(msg)
    return _ln(tot @ co_w + co_b, n_g, n_b)


def _mid_body(g_ref, s_ref, num_ref, g2_ref, m_ref, deg_ref,
              co_w, co_b, n_g, n_b,
              wi_w, wi_b, pg_w, pg_b, ng_g, ng_b, w_rel,
              g_o, y_o, tab_o, m_o):
    dinv_col = _dinv_col(deg_ref)
    h = _combine(g_ref[...], s_ref[...], num_ref[...], g2_ref[...],
                 m_ref[...], dinv_col, co_w[...], co_b[...], n_g[...], n_b[...])
    x = h @ wi_w[...] + wi_b[...]
    g, y, m = _dense_stage(x, dinv_col, w_rel[...], pg_w[...], pg_b[...],
                           ng_g[...], ng_b[...], tab_o)
    g_o[...] = g
    y_o[...] = y
    m_o[...] = m


def _post_body(g_ref, s_ref, num_ref, g2_ref, m_ref, deg_ref,
               co_w, co_b, n_g, n_b, h_o):
    dinv_col = _dinv_col(deg_ref)
    h = _combine(g_ref[...], s_ref[...], num_ref[...], g2_ref[...],
                 m_ref[...], dinv_col, co_w[...], co_b[...], n_g[...], n_b[...])
    h_o[...] = h * 0.5 * (1.0 + jax.lax.erf(h * (2.0 ** -0.5)))


_ND = jax.ShapeDtypeStruct((N, D), jnp.float32)
_DENSE_OUT = [_ND, _ND,
              jax.ShapeDtypeStruct((N, REL, D), jnp.float32),
              jax.ShapeDtypeStruct((1, D), jnp.float32)]

_pre_call = pl.pallas_call(_pre_body, out_shape=_DENSE_OUT)
_mid_call = pl.pallas_call(_mid_body, out_shape=_DENSE_OUT)
_post_call = pl.pallas_call(_post_body, out_shape=_ND)


def kernel(x, edge_index, idx, edge_type, params):
    src = edge_index[0]
    dst = edge_index[1]
    key2 = dst * 4 + edge_type

    deg_part = _deg_kernel(dst)

    l1, l2 = params["layers"]

    def row(v):
        return v.reshape(1, D)

    g1, y1, tab1, m1 = _pre_call(
        x, deg_part, params["proj_w"], row(params["proj_b"]),
        row(params["bn_g"]), row(params["bn_b"]),
        l1["wi_w"], row(l1["wi_b"]), l1["pg_w"], row(l1["pg_b"]),
        row(l1["ng_g"]), row(l1["ng_b"]), l1["w_rel"])

    s1, num1, g21 = _edge_kernel(tab1.reshape(N * REL, D), y1, src, key2, dst)

    g2_, y2, tab2, m2 = _mid_call(
        g1, s1, num1, g21, m1, deg_part,
        l1["co_w"], row(l1["co_b"]), row(l1["n_g"]), row(l1["n_b"]),
        l2["wi_w"], row(l2["wi_b"]), l2["pg_w"], row(l2["pg_b"]),
        row(l2["ng_g"]), row(l2["ng_b"]), l2["w_rel"])

    s2, num2, g22 = _edge_kernel(tab2.reshape(N * REL, D), y2, src, key2, dst)

    h = _post_call(g2_, s2, num2, g22, m2, deg_part,
                   l2["co_w"], row(l2["co_b"]), row(l2["n_g"]), row(l2["n_b"]))

    return _gather_kernel(h, idx)
